# Initial kernel scaffold; baseline (speedup 1.0000x reference)
import functools

import jax
import jax.numpy as jnp
from jax import lax
from jax.experimental import pallas as pl
from jax.experimental.pallas import tpu as pltpu
from jax.experimental.pallas import tpu_sc as plsc

N_NODES = 10000
N_EDGES = 320000
D_IN = 128
D_HID = 128
N_CLASSES = 40

NC = 2
NS = 16
NW = NC * NS
CHUNK = 128
E_PAD = 327680
CHUNKS_PER_W = E_PAD // (NW * CHUNK)
STRIPE = N_NODES // NS
ZROWS = 125


def _make_spmm(D):
  mesh = plsc.VectorSubcoreMesh(core_axis_name="c", subcore_axis_name="s")

  @functools.partial(
      pl.kernel,
      out_type=jax.ShapeDtypeStruct((NC, N_NODES, D), jnp.float32),
      mesh=mesh,
      scratch_types=[
          pltpu.VMEM((CHUNKS_PER_W, CHUNK), jnp.int32),
          pltpu.VMEM((CHUNKS_PER_W, CHUNK), jnp.int32),
          pltpu.VMEM((CHUNKS_PER_W, CHUNK), jnp.float32),
          pltpu.VMEM((CHUNK, D), jnp.float32),
          pltpu.VMEM((ZROWS, D), jnp.float32),
          pltpu.VMEM_SHARED((N_NODES, D), jnp.float32),
          pltpu.SemaphoreType.DMA,
          pltpu.SemaphoreType.DMA,
      ],
  )
  def spmm(y_hbm, src_hbm, dst_hbm, ew_hbm, out_hbm,
           src_v, dst_v, ew_v, rows_v, zbuf_v, acc, gsem, ssem):
    c = lax.axis_index("c")
    s = lax.axis_index("s")
    w = c * NS + s

    def zbody(i, carry):
      for cb in range(D // 16):
        zbuf_v[i, pl.ds(cb * 16, 16)] = jnp.zeros((16,), jnp.float32)
      return carry
    lax.fori_loop(0, ZROWS, zbody, 0)
    for r in range(STRIPE // ZROWS):
      pltpu.sync_copy(zbuf_v, acc.at[pl.ds(s * STRIPE + r * ZROWS, ZROWS)])

    pltpu.sync_copy(src_hbm.at[w], src_v)
    pltpu.sync_copy(dst_hbm.at[w], dst_v)
    pltpu.sync_copy(ew_hbm.at[w], ew_v)

    plsc.subcore_barrier()

    def body(j, carry):
      pltpu.async_copy(y_hbm.at[src_v.at[j]], rows_v, gsem).wait()
      for g in range(CHUNK // 16):
        ewg = ew_v[j, pl.ds(g * 16, 16)]
        for t in range(16):
          i = g * 16 + t
          wsc = ewg[t]
          for cb in range(D // 16):
            sl = pl.ds(cb * 16, 16)
            rows_v[i, sl] = rows_v[i, sl] * wsc
      pltpu.async_copy(rows_v, acc.at[dst_v.at[j]], ssem, add=True).wait()
      return carry
    lax.fori_loop(0, CHUNKS_PER_W, body, 0)

    plsc.subcore_barrier()
    pltpu.sync_copy(acc.at[pl.ds(s * STRIPE, STRIPE)],
                    out_hbm.at[c, pl.ds(s * STRIPE, STRIPE)])

  return spmm


_spmm_128 = _make_spmm(128)
_spmm_48 = _make_spmm(48)


def _mm1_body(x_ref, w_ref, o_ref):
  o_ref[...] = jnp.dot(x_ref[...], w_ref[...],
                       preferred_element_type=jnp.float32)


def _fuse_body(p_ref, b1_ref, w2_ref, o_ref):
  h = jnp.maximum(p_ref[0] + p_ref[1] + b1_ref[...][None, :], 0.0)
  o_ref[...] = jnp.dot(h, w2_ref[...], preferred_element_type=jnp.float32)


def _final_body(q_ref, b2_ref, o_ref):
  o_ref[...] = q_ref[0] + q_ref[1] + b2_ref[...][None, :]


@jax.jit
def kernel(in_feat, edge_index, edge_weight, W1, b1, W2, b2):
  src = edge_index[0].astype(jnp.int32)
  dst = edge_index[1].astype(jnp.int32)
  ew = edge_weight.astype(jnp.float32)

  npad = E_PAD - src.shape[0]
  pad_idx = (jnp.arange(npad, dtype=jnp.int32) * 13) % N_NODES
  src = jnp.concatenate([src, pad_idx]).reshape(NW, CHUNKS_PER_W, CHUNK)
  dst = jnp.concatenate([dst, pad_idx]).reshape(NW, CHUNKS_PER_W, CHUNK)
  ew = jnp.concatenate([ew, jnp.zeros((npad,), jnp.float32)])
  ew = ew.reshape(NW, CHUNKS_PER_W, CHUNK)

  y1 = pl.pallas_call(
      _mm1_body,
      out_shape=jax.ShapeDtypeStruct((N_NODES, D_HID), jnp.float32),
  )(in_feat, W1)
  p1 = _spmm_128(y1, src, dst, ew)

  w2p = jnp.pad(W2, ((0, 0), (0, 8)))
  y2 = pl.pallas_call(
      _fuse_body,
      out_shape=jax.ShapeDtypeStruct((N_NODES, 48), jnp.float32),
  )(p1, b1, w2p)
  p2 = _spmm_48(y2, src, dst, ew)

  b2p = jnp.pad(b2, (0, 8))
  outp = pl.pallas_call(
      _final_body,
      out_shape=jax.ShapeDtypeStruct((N_NODES, 48), jnp.float32),
  )(p2, b2p)
  return outp[:, :N_CLASSES]


# speedup vs baseline: 11.7808x; 11.7808x over previous
import functools

import jax
import jax.numpy as jnp
from jax import lax
from jax.experimental import pallas as pl
from jax.experimental.pallas import tpu as pltpu
from jax.experimental.pallas import tpu_sc as plsc

N_NODES = 10000
D_HID = 128
N_CLASSES = 40

NC = 2
NS = 16
CHUNK = 128
E_PAD = 327680
N_ACC = 10240
STRIPE = N_ACC // NS
ZROWS = 128
N_CHUNKS = E_PAD // (NS * CHUNK)

NBUF = 3
NSEM = 2


def _zero_acc(zbuf_v, acc, s, D):
  def zbody(i, carry):
    for cb in range(D // 16):
      zbuf_v[i, pl.ds(cb * 16, 16)] = jnp.zeros((16,), jnp.float32)
    return carry
  lax.fori_loop(0, ZROWS, zbody, 0)
  for r in range(STRIPE // ZROWS):
    pltpu.sync_copy(zbuf_v, acc.at[pl.ds(s * STRIPE + r * ZROWS, ZROWS)])


def _scale_rows(rows, b, ew_v, j, D):
  for g in range(CHUNK // 16):
    ewg = ew_v[j, pl.ds(g * 16, 16)]
    for t in range(16):
      i = g * 16 + t
      wv = lax.gather(
          ewg, jnp.full((16, 1), t, jnp.int32),
          dimension_numbers=lax.GatherDimensionNumbers(
              offset_dims=(), collapsed_slice_dims=(0,),
              start_index_map=(0,)),
          slice_sizes=(1,),
          mode=lax.GatherScatterMode.PROMISE_IN_BOUNDS)
      for cb in range(D // 16):
        sl = pl.ds(cb * 16, 16)
        rows[b, i, sl] = rows[b, i, sl] * wv


IBUF = 4


def _edge_loop(y_gather, n_chunks, src_hbm, dst_hbm, ew_hbm, s,
               src_w, dst_w, ew_w, rows, sems, isems, acc, D):
  def idx_fetch(j):
    sl = j % IBUF
    sem = isems.at[j % NSEM]
    pltpu.async_copy(src_hbm.at[s].at[j], src_w.at[sl], sem)
    pltpu.async_copy(dst_hbm.at[s].at[j], dst_w.at[sl], sem)
    pltpu.async_copy(ew_hbm.at[s].at[j], ew_w.at[sl], sem)

  def idx_wait(j):
    sl = j % IBUF
    sem = isems.at[j % NSEM]
    pltpu.make_async_copy(src_hbm.at[s].at[j], src_w.at[sl], sem).wait()
    pltpu.make_async_copy(dst_hbm.at[s].at[j], dst_w.at[sl], sem).wait()
    pltpu.make_async_copy(ew_hbm.at[s].at[j], ew_w.at[sl], sem).wait()

  def gather_issue(j):
    pltpu.async_copy(y_gather(src_w.at[j % IBUF]), rows.at[j % NBUF],
                     sems.at[j % NSEM])

  def gather_wait(j):
    pltpu.make_async_copy(y_gather(src_w.at[j % IBUF]), rows.at[j % NBUF],
                          sems.at[j % NSEM]).wait()

  def scatter_issue(j):
    pltpu.async_copy(rows.at[j % NBUF], acc.at[dst_w.at[j % IBUF]],
                     sems.at[NSEM + j % NSEM], add=True)

  def scatter_wait(j):
    pltpu.make_async_copy(rows.at[j % NBUF], acc.at[dst_w.at[j % IBUF]],
                          sems.at[NSEM + j % NSEM]).wait()

  idx_fetch(0)
  idx_fetch(1)
  idx_wait(0)
  gather_issue(0)
  idx_fetch(2)
  idx_wait(1)
  gather_issue(1)
  idx_fetch(3)

  def body(j, carry):
    gather_wait(j)
    @pl.when(j >= 1)
    def _():
      scatter_wait(j - 1)
      @pl.when(j + 3 < n_chunks)
      def _():
        idx_fetch(j + 3)
    @pl.when(j + 2 < n_chunks)
    def _():
      idx_wait(j + 2)
      gather_issue(j + 2)
    _scale_rows(rows, j % NBUF, ew_w, j % IBUF, D)
    scatter_issue(j)
    return carry
  lax.fori_loop(0, n_chunks, body, 0)
  scatter_wait(n_chunks - 1)


def _make_spmm(D):
  scratch = [
      pltpu.VMEM((IBUF, CHUNK), jnp.int32),
      pltpu.VMEM((IBUF, CHUNK), jnp.int32),
      pltpu.VMEM((IBUF, CHUNK), jnp.float32),
      pltpu.VMEM((NBUF, CHUNK, D), jnp.float32),
      pltpu.VMEM((ZROWS, D), jnp.float32),
      pltpu.VMEM_SHARED((N_ACC, D), jnp.float32),
      pltpu.SemaphoreType.DMA((2 * NSEM,)),
      pltpu.SemaphoreType.DMA((NSEM,)),
  ]

  @functools.partial(
      pl.kernel,
      out_type=jax.ShapeDtypeStruct((NC, N_ACC, D), jnp.float32),
      mesh=plsc.VectorSubcoreMesh(core_axis_name="c", subcore_axis_name="s"),
      scratch_types=scratch,
      compiler_params=pltpu.CompilerParams(use_tc_tiling_on_sc=False),
  )
  def spmm(y_hbm, src_hbm, dst_hbm, ew_hbm, out_hbm,
           src_w, dst_w, ew_w, rows, zbuf_v, acc, sems, isems):
    c = lax.axis_index("c")
    s = lax.axis_index("s")

    _zero_acc(zbuf_v, acc, s, D)

    plsc.subcore_barrier()

    _edge_loop(lambda idx: y_hbm.at[c].at[idx], N_CHUNKS,
               src_hbm, dst_hbm, ew_hbm, s,
               src_w, dst_w, ew_w, rows, sems, isems, acc, D)

    plsc.subcore_barrier()
    pltpu.sync_copy(acc.at[pl.ds(s * STRIPE, STRIPE)],
                    out_hbm.at[c, pl.ds(s * STRIPE, STRIPE)])

  return spmm


_spmm_l1 = _make_spmm(64)
_spmm_l2 = _make_spmm(32)


def _mm1_body(x_ref, w_ref, o_ref):
  y = jnp.dot(x_ref[...], w_ref[...], preferred_element_type=jnp.float32)
  o_ref[0] = y[:, :64]
  o_ref[1] = y[:, 64:]


def _fuse_body(p_ref, b1_ref, w2_ref, o_ref):
  b1 = b1_ref[...]
  h0 = jnp.maximum(p_ref[0] + b1[None, :64], 0.0)
  h1 = jnp.maximum(p_ref[1] + b1[None, 64:], 0.0)
  w2 = w2_ref[...]
  for c in range(NC):
    o_ref[c] = (
        jnp.dot(h0, w2[:64, c * 32:(c + 1) * 32],
                preferred_element_type=jnp.float32)
        + jnp.dot(h1, w2[64:, c * 32:(c + 1) * 32],
                  preferred_element_type=jnp.float32))


def _final_body(q_ref, b2_ref, o_ref):
  o_ref[...] = (jnp.concatenate([q_ref[0], q_ref[1][:, :16]], axis=1)
                + b2_ref[...][None, :])


@jax.jit
def kernel(in_feat, edge_index, edge_weight, W1, b1, W2, b2):
  src = edge_index[0].astype(jnp.int32)
  dst = edge_index[1].astype(jnp.int32)
  ew = edge_weight.astype(jnp.float32)

  npad = E_PAD - src.shape[0]
  pad_idx = (jnp.arange(npad, dtype=jnp.int32) * 13) % N_NODES
  src = jnp.concatenate([src, pad_idx]).reshape(NS, N_CHUNKS, CHUNK)
  dst = jnp.concatenate([dst, pad_idx]).reshape(NS, N_CHUNKS, CHUNK)
  ew = jnp.concatenate([ew, jnp.zeros((npad,), jnp.float32)])
  ew = ew.reshape(NS, N_CHUNKS, CHUNK)

  y1s = pl.pallas_call(
      _mm1_body,
      out_shape=jax.ShapeDtypeStruct((NC, N_NODES, 64), jnp.float32),
  )(in_feat, W1)
  p1 = _spmm_l1(y1s, src, dst, ew)

  w2p = jnp.pad(W2, ((0, 0), (0, 24)))
  y2s = pl.pallas_call(
      _fuse_body,
      out_shape=jax.ShapeDtypeStruct((NC, N_ACC, 32), jnp.float32),
  )(p1, b1, w2p)
  p2 = _spmm_l2(y2s, src, dst, ew)

  b2p = jnp.pad(b2, (0, 8))
  outp = pl.pallas_call(
      _final_body,
      out_shape=jax.ShapeDtypeStruct((N_ACC, 48), jnp.float32),
  )(p2, b2p)
  return outp[:N_NODES, :N_CLASSES]
